# Initial kernel scaffold; baseline (speedup 1.0000x reference)
import jax
import jax.numpy as jnp
from jax.experimental import pallas as pl


def kernel(q, k, v, Wq, Wk, Wv, sample_idx):
    raise NotImplementedError("write your pallas kernel here")


# speedup vs baseline: 7.4658x; 7.4658x over previous
import functools

import jax
import jax.numpy as jnp
import numpy as np
from jax.experimental import pallas as pl

HIDDEN = 768
NUM_HEADS = 12
FACTOR = 5


def _proj_kernel(q_ref, k_ref, v_ref, wq_ref, wk_ref, wv_ref,
                 qp_ref, kp_ref, vp_ref):
    qp_ref[...] = jnp.dot(q_ref[...], wq_ref[...],
                          preferred_element_type=jnp.float32)
    kp_ref[...] = jnp.dot(k_ref[...], wk_ref[...],
                          preferred_element_type=jnp.float32)
    vp_ref[...] = jnp.dot(v_ref[...], wv_ref[...],
                          preferred_element_type=jnp.float32)


def _project(q2, k2, v2, Wq, Wk, Wv, tile=512):
    BL, D = q2.shape
    grid = (BL // tile,)
    x_spec = pl.BlockSpec((tile, D), lambda i: (i, 0))
    w_spec = pl.BlockSpec((D, HIDDEN), lambda i: (0, 0))
    o_spec = pl.BlockSpec((tile, HIDDEN), lambda i: (i, 0))
    out_shape = jax.ShapeDtypeStruct((BL, HIDDEN), jnp.float32)
    return pl.pallas_call(
        _proj_kernel,
        grid=grid,
        in_specs=[x_spec, x_spec, x_spec, w_spec, w_spec, w_spec],
        out_specs=[o_spec, o_spec, o_spec],
        out_shape=[out_shape, out_shape, out_shape],
    )(q2, k2, v2, Wq, Wk, Wv)


def _count_kernel(idxt_ref, ct_ref, *, L, u):
    jidx = jax.lax.broadcasted_iota(jnp.int32, (L, L), 0)
    acc = jnp.zeros((L, L), jnp.float32)
    for s in range(u):
        row = idxt_ref[s:s + 1, :]
        acc = acc + (jidx == row).astype(jnp.float32)
    ct_ref[...] = acc


def _build_counts(sample_idx, L, u):
    idxt = sample_idx.T.reshape(u, L)
    return pl.pallas_call(
        functools.partial(_count_kernel, L=L, u=u),
        out_shape=jax.ShapeDtypeStruct((L, L), jnp.float32),
    )(idxt)


def _attn_kernel(qh_ref, kh_ref, vh_ref, ct_ref, out_ref, *, L, u, E, scale):
    qh = qh_ref[0]
    kh = kh_ref[0]
    vh = vh_ref[0]
    ct = ct_ref[...]

    st = jax.lax.dot_general(kh, qh, (((1,), (1,)), ((), ())),
                             preferred_element_type=jnp.float32)

    msum = jnp.sum(st * ct, axis=0, keepdims=True)
    mmax = jnp.max(jnp.where(ct > 0.0, st, -jnp.inf),
                   axis=0, keepdims=True)
    m = mmax - msum * (1.0 / L)

    lane = jax.lax.broadcasted_iota(jnp.int32, (1, L), 1)
    rows = []
    masks = []
    for _ in range(u):
        cur = jnp.max(m)
        pos = jnp.min(jnp.where(m == cur, lane, L))
        hit = lane == pos
        rows.append(hit.astype(jnp.float32))
        masks.append((lane > pos).astype(jnp.float32))
        m = jnp.where(hit, -jnp.inf, m)
    P = jnp.concatenate(rows, axis=0)
    causal = jnp.concatenate(masks, axis=0)

    qr = jnp.dot(P, qh, preferred_element_type=jnp.float32)
    scores = jax.lax.dot_general(qr, kh, (((1,), (1,)), ((), ())),
                                 preferred_element_type=jnp.float32)
    scores = scores * scale
    scores = jnp.where(causal > 0.0, -jnp.inf, scores)
    smax = jnp.max(scores, axis=1, keepdims=True)
    e = jnp.exp(scores - smax)
    attn = e / jnp.sum(e, axis=1, keepdims=True)
    upd = jnp.dot(attn, vh, preferred_element_type=jnp.float32)

    C = 256
    sub = jax.lax.broadcasted_iota(jnp.int32, (C, C), 0)
    lan = jax.lax.broadcasted_iota(jnp.int32, (C, C), 1)
    tri = (sub >= lan).astype(jnp.float32)
    chunks = []
    carry = jnp.zeros((1, E), jnp.float32)
    for c in range(L // C):
        vc = vh[c * C:(c + 1) * C, :]
        chunks.append(jnp.dot(tri, vc, preferred_element_type=jnp.float32)
                      + carry)
        carry = carry + jnp.sum(vc, axis=0, keepdims=True)
    ctx = jnp.concatenate(chunks, axis=0)

    selcol = jax.lax.dot_general(P, jnp.ones((u, 1), jnp.float32),
                                 (((0,), (0,)), ((), ())),
                                 preferred_element_type=jnp.float32)
    scat = jax.lax.dot_general(P, upd, (((0,), (0,)), ((), ())),
                               preferred_element_type=jnp.float32)
    out_ref[0] = ctx * (1.0 - selcol) + scat


def _attention(qh, kh, vh, ct, L, u, E, scale):
    BH = qh.shape[0]
    blk = pl.BlockSpec((1, L, E), lambda i: (i, 0, 0))
    ct_spec = pl.BlockSpec((L, L), lambda i: (0, 0))
    return pl.pallas_call(
        functools.partial(_attn_kernel, L=L, u=u, E=E, scale=scale),
        grid=(BH,),
        in_specs=[blk, blk, blk, ct_spec],
        out_specs=blk,
        out_shape=jax.ShapeDtypeStruct((BH, L, E), jnp.float32),
    )(qh, kh, vh, ct)


def kernel(q, k, v, Wq, Wk, Wv, sample_idx):
    B, L, D = q.shape
    H = NUM_HEADS
    E = HIDDEN // H
    u = min(FACTOR * int(np.ceil(np.log(L))), L)
    scale = float(1.0 / np.sqrt(HIDDEN // H))

    qp, kp, vp = _project(q.reshape(B * L, D), k.reshape(B * L, D),
                          v.reshape(B * L, D), Wq, Wk, Wv)
    qh = qp.reshape(B * H, L, E)
    kh = kp.reshape(B * H, L, E)
    vh = vp.reshape(B * H, L, E)

    ct = _build_counts(sample_idx, L, u)
    ctx = _attention(qh, kh, vh, ct, L, u, E, scale)
    return ctx.reshape(B, L, HIDDEN)
